# baseline (speedup 1.0000x reference)
import functools

import jax
import jax.numpy as jnp
from jax import lax
from jax.experimental import pallas as pl
from jax.experimental.pallas import tpu as pltpu
from jax.experimental.pallas import tpu_sc as plsc

B = 1024
N = 128
F = 9
H = 64
W = H // 2
HN = N // 2
HF = HN * F
ATOM_MAX = 100000
DEG_MAX = 511
NW = 32
BPW = B // NW
SC_B = 32
G = F
HIMASK = -65536


def _sc_body(x_hbm, indeg_hbm, outdeg_hbm, atom_hbm, indt_hbm, outdt_hbm,
             tok_hbm, out_hbm,
             xs_v, dis_v, dos_v, rows_a, rows_b, ind_a, ind_b, outd_a,
             outd_b, out_a, out_b, tok_v, sem_a, sem_b, sem_oa, sem_ob):
  wid = lax.axis_index("s") * 2 + lax.axis_index("c")
  pltpu.sync_copy(tok_hbm, tok_v)

  def clamp_and_fire(bl, h, rows_v, ind_v, outd_v, sem):
    for g in range(G):
      for j in range(HN // 16):
        s = pl.ds(j * 16, 16)
        v = xs_v[bl, h * G + g, s]
        xs_v[bl, h * G + g, s] = jnp.minimum(jnp.maximum(v, 0), ATOM_MAX)
    for j in range(HN // 16):
      s = pl.ds(h * HN + j * 16, 16)
      v = dis_v[bl, s]
      dis_v[bl, s] = jnp.minimum(jnp.maximum(v, 0), DEG_MAX)
      w = dos_v[bl, s]
      dos_v[bl, s] = jnp.minimum(jnp.maximum(w, 0), DEG_MAX)
    for g in range(G):
      pltpu.async_copy(atom_hbm.at[xs_v.at[bl, h * G + g]],
                       rows_v.at[pl.ds(g * HN, HN)], sem)
    pltpu.async_copy(indt_hbm.at[dis_v.at[bl, pl.ds(h * HN, HN)]],
                     ind_v, sem)
    pltpu.async_copy(outdt_hbm.at[dos_v.at[bl, pl.ds(h * HN, HN)]],
                     outd_v, sem)

  def drain(rows_v, ind_v, outd_v, sem):
    pltpu.make_async_copy(atom_hbm.at[pl.ds(0, HF)], rows_v, sem).wait()
    pltpu.make_async_copy(atom_hbm.at[pl.ds(0, HN)], ind_v, sem).wait()
    pltpu.make_async_copy(atom_hbm.at[pl.ds(0, HN)], outd_v, sem).wait()

  def unpack_lo(v):
    return plsc.bitcast(jnp.left_shift(v, 16), jnp.float32)

  def unpack_hi(v):
    return plsc.bitcast(jnp.bitwise_and(v, HIMASK), jnp.float32)

  def accumulate(rows_v, ind_v, outd_v, out_v, row0):
    even = lax.iota(jnp.int32, 16) * 2

    def node_body(c, carry):
      base = c * F
      for j in range(2):
        s = pl.ds(j * 16, 16)
        iv = ind_v[c, s]
        ov = outd_v[c, s]
        lo = unpack_lo(iv) + unpack_lo(ov)
        hi = unpack_hi(iv) + unpack_hi(ov)
        for k in range(F):
          v = rows_v[base + k, s]
          lo = lo + unpack_lo(v)
          hi = hi + unpack_hi(v)
        row = out_v.at[c + row0]
        plsc.store_scatter(row, [even + (j * 32)], lo)
        plsc.store_scatter(row, [even + (j * 32 + 1)], hi)
      return carry
    lax.fori_loop(0, HN, node_body, 0, unroll=4)

  b0 = wid * BPW
  pltpu.sync_copy(x_hbm.at[pl.ds(b0, SC_B)], xs_v)
  pltpu.sync_copy(indeg_hbm.at[pl.ds(b0, SC_B)], dis_v)
  pltpu.sync_copy(outdeg_hbm.at[pl.ds(b0, SC_B)], dos_v)
  clamp_and_fire(0, 0, rows_a, ind_a, outd_a, sem_a)

  def batch_body(bl, carry):
    b = b0 + bl
    clamp_and_fire(bl, 1, rows_b, ind_b, outd_b, sem_b)
    drain(rows_a, ind_a, outd_a, sem_a)
    @pl.when(bl > 0)
    def _():
      pltpu.make_async_copy(out_a, out_hbm.at[b, pl.ds(0, HN + 1)],
                            sem_oa).wait()
    for j in range(H // 16):
      out_a[0, pl.ds(j * 16, 16)] = tok_v[0, pl.ds(j * 16, 16)]
    accumulate(rows_a, ind_a, outd_a, out_a, 1)
    pltpu.async_copy(out_a, out_hbm.at[b, pl.ds(0, HN + 1)], sem_oa)
    @pl.when(bl < SC_B - 1)
    def _():
      clamp_and_fire(bl + 1, 0, rows_a, ind_a, outd_a, sem_a)
    drain(rows_b, ind_b, outd_b, sem_b)
    @pl.when(bl > 0)
    def _():
      pltpu.make_async_copy(out_b, out_hbm.at[b, pl.ds(HN + 1, HN)],
                            sem_ob).wait()
    accumulate(rows_b, ind_b, outd_b, out_b, 0)
    pltpu.async_copy(out_b, out_hbm.at[b, pl.ds(HN + 1, HN)], sem_ob)
    return carry

  lax.fori_loop(0, SC_B, batch_body, 0)
  pltpu.make_async_copy(out_a, out_hbm.at[b0, pl.ds(0, HN + 1)],
                        sem_oa).wait()
  pltpu.make_async_copy(out_b, out_hbm.at[b0, pl.ds(HN + 1, HN)],
                        sem_ob).wait()


@jax.jit
def _run(x1, indeg, outdeg, atom_p, indt_p, outdt_p, tok):
  mesh = plsc.VectorSubcoreMesh(core_axis_name="c", subcore_axis_name="s")
  fn = functools.partial(
      pl.kernel,
      mesh=mesh,
      compiler_params=pltpu.CompilerParams(use_tc_tiling_on_sc=False,
                                           needs_layout_passes=False),
      out_type=jax.ShapeDtypeStruct((B, N + 1, H), jnp.float32),
      scratch_types=[
          pltpu.VMEM((SC_B, 2 * G, HN), jnp.int32),
          pltpu.VMEM((SC_B, N), jnp.int32),
          pltpu.VMEM((SC_B, N), jnp.int32),
          pltpu.VMEM((HF, W), jnp.int32),
          pltpu.VMEM((HF, W), jnp.int32),
          pltpu.VMEM((HN, W), jnp.int32),
          pltpu.VMEM((HN, W), jnp.int32),
          pltpu.VMEM((HN, W), jnp.int32),
          pltpu.VMEM((HN, W), jnp.int32),
          pltpu.VMEM((HN + 1, H), jnp.float32),
          pltpu.VMEM((HN, H), jnp.float32),
          pltpu.VMEM((1, H), jnp.float32),
          pltpu.SemaphoreType.DMA,
          pltpu.SemaphoreType.DMA,
          pltpu.SemaphoreType.DMA,
          pltpu.SemaphoreType.DMA,
      ],
  )(_sc_body)
  return fn(x1, indeg, outdeg, atom_p, indt_p, outdt_p, tok)


def _pack_table(t):
  v = t.shape[0]
  return lax.bitcast_convert_type(
      t.astype(jnp.bfloat16).reshape(v, W, 2), jnp.int32)


def kernel(x, in_degree, out_degree, atom_table, in_deg_table,
           out_deg_table, graph_token):
  x1 = x.astype(jnp.int32).reshape(B, 2 * G, HN)
  return _run(x1, in_degree.astype(jnp.int32), out_degree.astype(jnp.int32),
              _pack_table(atom_table), _pack_table(in_deg_table),
              _pack_table(out_deg_table), graph_token)


# speedup vs baseline: 1.4753x; 1.4753x over previous
import functools

import jax
import jax.numpy as jnp
from jax import lax
from jax.experimental import pallas as pl
from jax.experimental.pallas import tpu as pltpu
from jax.experimental.pallas import tpu_sc as plsc

B = 1024
N = 128
F = 9
H = 64
W = H // 2
HN = N // 2
HF = HN * F
ATOM_MAX = 100000
DEG_MAX = 511
NW = 32
BPW = B // NW
SC_B = 32
G = F
HIMASK = -65536


def _sc_body(x_hbm, indeg_hbm, outdeg_hbm, atom_hbm, indt_hbm, outdt_hbm,
             tok_hbm, out_hbm,
             xs_v, dis_v, dos_v, rows_a, rows_b, ind_a, ind_b, outd_a,
             outd_b, out_a, out_b, tok_v, sem_a, sem_b, sem_oa, sem_ob):
  wid = lax.axis_index("s") * 2 + lax.axis_index("c")
  pltpu.sync_copy(tok_hbm, tok_v)

  def clamp_and_fire(bl, h, rows_v, ind_v, outd_v, sem):
    for g in range(G):
      for j in range(HN // 16):
        s = pl.ds(j * 16, 16)
        v = xs_v[bl, h * G + g, s]
        xs_v[bl, h * G + g, s] = jnp.minimum(jnp.maximum(v, 0), ATOM_MAX)
    for j in range(HN // 16):
      s = pl.ds(h * HN + j * 16, 16)
      v = dis_v[bl, s]
      dis_v[bl, s] = jnp.minimum(jnp.maximum(v, 0), DEG_MAX)
      w = dos_v[bl, s]
      dos_v[bl, s] = jnp.minimum(jnp.maximum(w, 0), DEG_MAX)
    for g in range(G):
      pltpu.async_copy(atom_hbm.at[xs_v.at[bl, h * G + g]],
                       rows_v.at[pl.ds(g * HN, HN)], sem)
    pltpu.async_copy(indt_hbm.at[dis_v.at[bl, pl.ds(h * HN, HN)]],
                     ind_v, sem)
    pltpu.async_copy(outdt_hbm.at[dos_v.at[bl, pl.ds(h * HN, HN)]],
                     outd_v, sem)

  def drain(rows_v, ind_v, outd_v, sem):
    pltpu.make_async_copy(atom_hbm.at[pl.ds(0, HF)], rows_v, sem).wait()
    pltpu.make_async_copy(atom_hbm.at[pl.ds(0, HN)], ind_v, sem).wait()
    pltpu.make_async_copy(atom_hbm.at[pl.ds(0, HN)], outd_v, sem).wait()

  def unpack_lo(v):
    return plsc.bitcast(jnp.left_shift(v, 16), jnp.float32)

  def unpack_hi(v):
    return plsc.bitcast(jnp.bitwise_and(v, HIMASK), jnp.float32)

  def accumulate(rows_v, ind_v, outd_v, out_v, row0):
    def node_body(c, carry):
      base = c * F
      for j in range(2):
        s = pl.ds(j * 16, 16)
        iv = ind_v[c, s]
        ov = outd_v[c, s]
        lo = unpack_lo(iv) + unpack_lo(ov)
        hi = unpack_hi(iv) + unpack_hi(ov)
        for k in range(F):
          v = rows_v[base + k, s]
          lo = lo + unpack_lo(v)
          hi = hi + unpack_hi(v)
        out_v[pl.ds((c + row0) * H + j * 32, 16)] = lo
        out_v[pl.ds((c + row0) * H + j * 32 + 16, 16)] = hi
      return carry
    lax.fori_loop(0, HN, node_body, 0, unroll=4)

  b0 = wid * BPW
  pltpu.sync_copy(x_hbm.at[pl.ds(b0, SC_B)], xs_v)
  pltpu.sync_copy(indeg_hbm.at[pl.ds(b0, SC_B)], dis_v)
  pltpu.sync_copy(outdeg_hbm.at[pl.ds(b0, SC_B)], dos_v)
  clamp_and_fire(0, 0, rows_a, ind_a, outd_a, sem_a)

  def batch_body(bl, carry):
    b = b0 + bl
    clamp_and_fire(bl, 1, rows_b, ind_b, outd_b, sem_b)
    drain(rows_a, ind_a, outd_a, sem_a)
    @pl.when(bl > 0)
    def _():
      pltpu.make_async_copy(out_a, out_hbm.at[pl.ds(0, (HN + 1) * H)],
                            sem_oa).wait()
    for j in range(H // 16):
      out_a[pl.ds(j * 16, 16)] = tok_v[0, pl.ds(j * 16, 16)]
    accumulate(rows_a, ind_a, outd_a, out_a, 1)
    pltpu.async_copy(out_a, out_hbm.at[pl.ds(b * ((N + 1) * H), (HN + 1) * H)],
                     sem_oa)
    @pl.when(bl < SC_B - 1)
    def _():
      clamp_and_fire(bl + 1, 0, rows_a, ind_a, outd_a, sem_a)
    drain(rows_b, ind_b, outd_b, sem_b)
    @pl.when(bl > 0)
    def _():
      pltpu.make_async_copy(out_b, out_hbm.at[pl.ds(0, HN * H)],
                            sem_ob).wait()
    accumulate(rows_b, ind_b, outd_b, out_b, 0)
    pltpu.async_copy(
        out_b, out_hbm.at[pl.ds(b * ((N + 1) * H) + (HN + 1) * H, HN * H)],
        sem_ob)
    return carry

  lax.fori_loop(0, SC_B, batch_body, 0)
  pltpu.make_async_copy(out_a, out_hbm.at[pl.ds(0, (HN + 1) * H)],
                        sem_oa).wait()
  pltpu.make_async_copy(out_b, out_hbm.at[pl.ds(0, HN * H)],
                        sem_ob).wait()


@jax.jit
def _run(x1, indeg, outdeg, atom_p, indt_p, outdt_p, tok):
  mesh = plsc.VectorSubcoreMesh(core_axis_name="c", subcore_axis_name="s")
  fn = functools.partial(
      pl.kernel,
      mesh=mesh,
      compiler_params=pltpu.CompilerParams(use_tc_tiling_on_sc=False,
                                           needs_layout_passes=False),
      out_type=jax.ShapeDtypeStruct((B * (N + 1) * H,), jnp.float32),
      scratch_types=[
          pltpu.VMEM((SC_B, 2 * G, HN), jnp.int32),
          pltpu.VMEM((SC_B, N), jnp.int32),
          pltpu.VMEM((SC_B, N), jnp.int32),
          pltpu.VMEM((HF, W), jnp.int32),
          pltpu.VMEM((HF, W), jnp.int32),
          pltpu.VMEM((HN, W), jnp.int32),
          pltpu.VMEM((HN, W), jnp.int32),
          pltpu.VMEM((HN, W), jnp.int32),
          pltpu.VMEM((HN, W), jnp.int32),
          pltpu.VMEM(((HN + 1) * H,), jnp.float32),
          pltpu.VMEM((HN * H,), jnp.float32),
          pltpu.VMEM((1, H), jnp.float32),
          pltpu.SemaphoreType.DMA,
          pltpu.SemaphoreType.DMA,
          pltpu.SemaphoreType.DMA,
          pltpu.SemaphoreType.DMA,
      ],
  )(_sc_body)
  return fn(x1, indeg, outdeg, atom_p, indt_p, outdt_p,
            tok).reshape(B, N + 1, H)


def _pack_table(t):
  v = t.shape[0]
  t4 = t.reshape(v, 2, 2, 16).transpose(0, 1, 3, 2)
  return lax.bitcast_convert_type(
      t4.astype(jnp.bfloat16).reshape(v, W, 2), jnp.int32)


def kernel(x, in_degree, out_degree, atom_table, in_deg_table,
           out_deg_table, graph_token):
  x1 = x.astype(jnp.int32).reshape(B, 2 * G, HN)
  return _run(x1, in_degree.astype(jnp.int32), out_degree.astype(jnp.int32),
              _pack_table(atom_table), _pack_table(in_deg_table),
              _pack_table(out_deg_table), graph_token)


# speedup vs baseline: 1.4827x; 1.0050x over previous
import functools

import jax
import jax.numpy as jnp
from jax import lax
from jax.experimental import pallas as pl
from jax.experimental.pallas import tpu as pltpu
from jax.experimental.pallas import tpu_sc as plsc

B = 1024
N = 128
F = 9
H = 64
W = H // 2
HN = N // 2
HF = HN * F
ATOM_MAX = 100000
DEG_MAX = 511
NW = 32
BPW = B // NW
SC_B = 32
G = F
HIMASK = -65536


def _sc_body(x_hbm, indeg_hbm, outdeg_hbm, atom_hbm, indt_hbm, outdt_hbm,
             tok_hbm, out_hbm,
             xs_v, dis_v, dos_v, rows_a, rows_b, ind_a, ind_b, outd_a,
             outd_b, out_a, out_b, tok_v, sem_a, sem_b, sem_oa, sem_ob):
  wid = lax.axis_index("s") * 2 + lax.axis_index("c")
  pltpu.sync_copy(tok_hbm, tok_v)

  def clamp_and_fire(bl, h, rows_v, ind_v, outd_v, sem):
    for g in range(G):
      for j in range(HN // 16):
        s = pl.ds(j * 16, 16)
        v = xs_v[bl, h * G + g, s]
        xs_v[bl, h * G + g, s] = jnp.minimum(jnp.maximum(v, 0), ATOM_MAX)
    for j in range(HN // 16):
      s = pl.ds(h * HN + j * 16, 16)
      v = dis_v[bl, s]
      dis_v[bl, s] = jnp.minimum(jnp.maximum(v, 0), DEG_MAX)
      w = dos_v[bl, s]
      dos_v[bl, s] = jnp.minimum(jnp.maximum(w, 0), DEG_MAX)
    for g in range(G):
      pltpu.async_copy(atom_hbm.at[xs_v.at[bl, h * G + g]],
                       rows_v.at[pl.ds(g * HN, HN)], sem)
    pltpu.async_copy(indt_hbm.at[dis_v.at[bl, pl.ds(h * HN, HN)]],
                     ind_v, sem)
    pltpu.async_copy(outdt_hbm.at[dos_v.at[bl, pl.ds(h * HN, HN)]],
                     outd_v, sem)

  def drain(rows_v, ind_v, outd_v, sem):
    pltpu.make_async_copy(atom_hbm.at[pl.ds(0, HF)], rows_v, sem).wait()
    pltpu.make_async_copy(atom_hbm.at[pl.ds(0, HN)], ind_v, sem).wait()
    pltpu.make_async_copy(atom_hbm.at[pl.ds(0, HN)], outd_v, sem).wait()

  def unpack_lo(v):
    return plsc.bitcast(jnp.left_shift(v, 16), jnp.float32)

  def unpack_hi(v):
    return plsc.bitcast(jnp.bitwise_and(v, HIMASK), jnp.float32)

  def accumulate(rows_v, ind_v, outd_v, out_v, row0):
    def node_body(c, carry):
      base = c * F
      for j in range(2):
        s = pl.ds(j * 16, 16)
        iv = ind_v[c, s]
        ov = outd_v[c, s]
        lo = unpack_lo(iv) + unpack_lo(ov)
        hi = unpack_hi(iv) + unpack_hi(ov)
        for k in range(F):
          v = rows_v[base + k, s]
          lo = lo + unpack_lo(v)
          hi = hi + unpack_hi(v)
        out_v[c + row0, pl.ds(j * 32, 16)] = lo
        out_v[c + row0, pl.ds(j * 32 + 16, 16)] = hi
      return carry
    lax.fori_loop(0, HN, node_body, 0, unroll=8)

  b0 = wid * BPW
  pltpu.sync_copy(x_hbm.at[pl.ds(b0, SC_B)], xs_v)
  pltpu.sync_copy(indeg_hbm.at[pl.ds(b0, SC_B)], dis_v)
  pltpu.sync_copy(outdeg_hbm.at[pl.ds(b0, SC_B)], dos_v)
  clamp_and_fire(0, 0, rows_a, ind_a, outd_a, sem_a)

  def batch_body(bl, carry):
    b = b0 + bl
    clamp_and_fire(bl, 1, rows_b, ind_b, outd_b, sem_b)
    drain(rows_a, ind_a, outd_a, sem_a)
    @pl.when(bl > 0)
    def _():
      pltpu.make_async_copy(out_a, out_hbm.at[b, pl.ds(0, HN + 1)],
                            sem_oa).wait()
    for j in range(H // 16):
      out_a[0, pl.ds(j * 16, 16)] = tok_v[0, pl.ds(j * 16, 16)]
    accumulate(rows_a, ind_a, outd_a, out_a, 1)
    pltpu.async_copy(out_a, out_hbm.at[b, pl.ds(0, HN + 1)], sem_oa)
    @pl.when(bl < SC_B - 1)
    def _():
      clamp_and_fire(bl + 1, 0, rows_a, ind_a, outd_a, sem_a)
    drain(rows_b, ind_b, outd_b, sem_b)
    @pl.when(bl > 0)
    def _():
      pltpu.make_async_copy(out_b, out_hbm.at[b, pl.ds(HN + 1, HN)],
                            sem_ob).wait()
    accumulate(rows_b, ind_b, outd_b, out_b, 0)
    pltpu.async_copy(out_b, out_hbm.at[b, pl.ds(HN + 1, HN)], sem_ob)
    return carry

  lax.fori_loop(0, SC_B, batch_body, 0)
  pltpu.make_async_copy(out_a, out_hbm.at[b0, pl.ds(0, HN + 1)],
                        sem_oa).wait()
  pltpu.make_async_copy(out_b, out_hbm.at[b0, pl.ds(HN + 1, HN)],
                        sem_ob).wait()


@jax.jit
def _run(x1, indeg, outdeg, atom_p, indt_p, outdt_p, tok):
  mesh = plsc.VectorSubcoreMesh(core_axis_name="c", subcore_axis_name="s")
  fn = functools.partial(
      pl.kernel,
      mesh=mesh,
      compiler_params=pltpu.CompilerParams(use_tc_tiling_on_sc=False,
                                           needs_layout_passes=False),
      out_type=jax.ShapeDtypeStruct((B, N + 1, H), jnp.float32),
      scratch_types=[
          pltpu.VMEM((SC_B, 2 * G, HN), jnp.int32),
          pltpu.VMEM((SC_B, N), jnp.int32),
          pltpu.VMEM((SC_B, N), jnp.int32),
          pltpu.VMEM((HF, W), jnp.int32),
          pltpu.VMEM((HF, W), jnp.int32),
          pltpu.VMEM((HN, W), jnp.int32),
          pltpu.VMEM((HN, W), jnp.int32),
          pltpu.VMEM((HN, W), jnp.int32),
          pltpu.VMEM((HN, W), jnp.int32),
          pltpu.VMEM((HN + 1, H), jnp.float32),
          pltpu.VMEM((HN, H), jnp.float32),
          pltpu.VMEM((1, H), jnp.float32),
          pltpu.SemaphoreType.DMA,
          pltpu.SemaphoreType.DMA,
          pltpu.SemaphoreType.DMA,
          pltpu.SemaphoreType.DMA,
      ],
  )(_sc_body)
  return fn(x1, indeg, outdeg, atom_p, indt_p, outdt_p, tok)


def _pack_table(t):
  v = t.shape[0]
  t4 = t.reshape(v, 2, 2, 16).transpose(0, 1, 3, 2)
  return lax.bitcast_convert_type(
      t4.astype(jnp.bfloat16).reshape(v, W, 2), jnp.int32)


def kernel(x, in_degree, out_degree, atom_table, in_deg_table,
           out_deg_table, graph_token):
  x1 = x.astype(jnp.int32).reshape(B, 2 * G, HN)
  return _run(x1, in_degree.astype(jnp.int32), out_degree.astype(jnp.int32),
              _pack_table(atom_table), _pack_table(in_deg_table),
              _pack_table(out_deg_table), graph_token)
